# Initial kernel scaffold; baseline (speedup 1.0000x reference)
import jax
import jax.numpy as jnp
from jax.experimental import pallas as pl


def kernel(x, edge_index, edge_attr, edge_weight, W_lin1, W_e1, b_e1, W_e2, b_e2, W_lin2, b_lin2):
    raise NotImplementedError("write your pallas kernel here")


# speedup vs baseline: 1.2771x; 1.2771x over previous
import functools
import math

import jax
import jax.numpy as jnp
from jax import lax
from jax.experimental import pallas as pl
from jax.experimental.pallas import tpu as pltpu
from jax.experimental.pallas import tpu_sc as plsc

N = 10000
E = 320000
D = 128
NG = 50
NF = 64
NFP = 128
CUTOFF = 5.0

NC = 2
NS = 16
L = 16
NW = NC * NS
EPW = E // NW
CHUNK = 80
NCHUNK = EPW // CHUNK
ZTILES = 10
ZPT = N // ZTILES
ZROWS = 200
ZREP = ZPT // ZROWS

EBLK = 4000


def _wfilt_body(attr_ref, w_ref, we1_ref, be1_ref, we2_ref, be2_ref, out_ref):
    attr = attr_ref[...]
    w = w_ref[...]
    step = CUTOFF / (NG - 1)
    offset = lax.broadcasted_iota(jnp.int32, (1, NG), 1).astype(jnp.float32) * step
    coeff = -0.5 / step ** 2
    dist = attr - offset
    smeared = jnp.exp(coeff * dist * dist)
    hgt = jnp.dot(smeared, we1_ref[...], preferred_element_type=jnp.float32)
    hgt = jnp.maximum(hgt + be1_ref[...], 0.0)
    wf = jnp.dot(hgt, we2_ref[...], preferred_element_type=jnp.float32)
    wf = wf + be2_ref[...]
    cc = 0.5 * (jnp.cos(w * (math.pi / CUTOFF)) + 1.0)
    out_ref[...] = wf * cc


_wfilt_call = pl.pallas_call(
    _wfilt_body,
    grid=(E // EBLK,),
    in_specs=[
        pl.BlockSpec((EBLK, 1), lambda i: (i, 0)),
        pl.BlockSpec((EBLK, 1), lambda i: (i, 0)),
        pl.BlockSpec((NG, NF), lambda i: (0, 0)),
        pl.BlockSpec((1, NF), lambda i: (0, 0)),
        pl.BlockSpec((NF, NFP), lambda i: (0, 0)),
        pl.BlockSpec((1, NFP), lambda i: (0, 0)),
    ],
    out_specs=pl.BlockSpec((EBLK, NFP), lambda i: (i, 0)),
    out_shape=jax.ShapeDtypeStruct((E, NFP), jnp.float32),
)


def _h_body(x_ref, w_ref, out_ref):
    out_ref[...] = jnp.dot(x_ref[...], w_ref[...],
                           preferred_element_type=jnp.float32)


_h_call = pl.pallas_call(
    _h_body,
    out_shape=jax.ShapeDtypeStruct((N, NFP), jnp.float32),
)


def _sc_body(h_hbm, wf_hbm, src_hbm, dst_hbm, out_hbm,
             agg_sh, src_v, dst_v, wf_v, rows_v, zero_v, sem):
    cid = lax.axis_index("c")
    sid = lax.axis_index("s")
    wid = sid * NC + cid

    @pl.when(sid < ZTILES)
    def _zero():
        def _zrow(i, carry):
            for j in range(NFP // L):
                zero_v[i, pl.ds(j * L, L)] = jnp.zeros((L,), jnp.float32)
            return carry
        lax.fori_loop(0, ZROWS, _zrow, 0)

        def _zcopy(k, carry):
            zbase = pl.multiple_of(sid * ZPT + k * ZROWS, 8)
            pltpu.sync_copy(zero_v, agg_sh.at[pl.ds(zbase, ZROWS)])
            return carry
        lax.fori_loop(0, ZREP, _zcopy, 0)
    plsc.subcore_barrier()

    def _chunk(c, carry):
        base = pl.multiple_of(wid * EPW + c * CHUNK, 8)
        pltpu.sync_copy(src_hbm.at[pl.ds(base, CHUNK)], src_v)
        pltpu.sync_copy(dst_hbm.at[pl.ds(base, CHUNK)], dst_v)
        pltpu.sync_copy(wf_hbm.at[pl.ds(base, CHUNK)], wf_v)
        pltpu.async_copy(h_hbm.at[src_v], rows_v, sem).wait()

        def _mul(i, icarry):
            for j in range(NF // L):
                s = pl.ds(j * L, L)
                rows_v[i, s] = rows_v[i, s] * wf_v[i, s]
            return icarry
        lax.fori_loop(0, CHUNK, _mul, 0)

        pltpu.sync_copy(rows_v, agg_sh.at[dst_v], add=True)
        return carry
    lax.fori_loop(0, NCHUNK, _chunk, 0)

    plsc.subcore_barrier()

    @pl.when(sid < ZTILES)
    def _copy_out():
        def _ocopy(k, carry):
            zbase = pl.multiple_of(sid * ZPT + k * ZROWS, 8)
            obase = pl.multiple_of(cid * N + sid * ZPT + k * ZROWS, 8)
            pltpu.sync_copy(agg_sh.at[pl.ds(zbase, ZROWS)],
                            out_hbm.at[pl.ds(obase, ZROWS)])
            return carry
        lax.fori_loop(0, ZREP, _ocopy, 0)


_sc_call = functools.partial(
    pl.kernel,
    mesh=plsc.VectorSubcoreMesh(core_axis_name="c", subcore_axis_name="s"),
    out_type=jax.ShapeDtypeStruct((NC * N, NFP), jnp.float32),
    scratch_types=[
        pltpu.VMEM_SHARED((N, NFP), jnp.float32),
        pltpu.VMEM((CHUNK,), jnp.int32),
        pltpu.VMEM((CHUNK,), jnp.int32),
        pltpu.VMEM((CHUNK, NFP), jnp.float32),
        pltpu.VMEM((CHUNK, NFP), jnp.float32),
        pltpu.VMEM((ZROWS, NFP), jnp.float32),
        pltpu.SemaphoreType.DMA,
    ],
)(_sc_body)


def _final_body(agg_ref, x_ref, w_ref, b_ref, out_ref):
    aggsum = agg_ref[:N, :NF] + agg_ref[N:, :NF]
    y = jnp.dot(aggsum, w_ref[...], preferred_element_type=jnp.float32)
    y = jnp.maximum(y + b_ref[...], 0.0)
    out_ref[...] = x_ref[...] + y


_final_call = pl.pallas_call(
    _final_body,
    out_shape=jax.ShapeDtypeStruct((N, D), jnp.float32),
)


def kernel(x, edge_index, edge_attr, edge_weight, W_lin1, W_e1, b_e1,
           W_e2, b_e2, W_lin2, b_lin2):
    src = edge_index[0].astype(jnp.int32)
    dst = edge_index[1].astype(jnp.int32)
    pad = jnp.zeros((NF, NFP - NF), jnp.float32)
    we2p = jnp.concatenate([W_e2, pad], axis=1)
    be2p = jnp.concatenate([b_e2, jnp.zeros((NFP - NF,), jnp.float32)])
    w1p = jnp.concatenate([W_lin1, jnp.zeros((D, NFP - NF), jnp.float32)],
                          axis=1)
    wfilt = _wfilt_call(edge_attr.reshape(E, 1), edge_weight.reshape(E, 1),
                        W_e1, b_e1.reshape(1, NF),
                        we2p, be2p.reshape(1, NFP))
    h = _h_call(x, w1p)
    agg2 = _sc_call(h, wfilt, src, dst)
    return _final_call(agg2, x, W_lin2, b_lin2.reshape(1, D))


# speedup vs baseline: 1.5243x; 1.1935x over previous
import functools
import math

import jax
import jax.numpy as jnp
from jax import lax
from jax.experimental import pallas as pl
from jax.experimental.pallas import tpu as pltpu
from jax.experimental.pallas import tpu_sc as plsc

N = 10000
E = 320000
D = 128
NG = 50
NF = 64
NFP = 128
CUTOFF = 5.0

NC = 2
NS = 16
L = 16
NW = NC * NS
EPW = E // NW
CHUNK = 80
NCHUNK = EPW // CHUNK
ZTILES = 10
ZPT = N // ZTILES
ZROWS = 200
ZREP = ZPT // ZROWS

ERB = 25
EBLK = ERB * 128
EGRID = E // EBLK


def _lanes_to_col(v):
    vb = jnp.broadcast_to(v.reshape(ERB, 1, 128), (ERB, 128, 128))
    vb = vb.reshape(EBLK, 128)
    lane = lax.broadcasted_iota(jnp.int32, (EBLK, 128), 1)
    pos = lax.broadcasted_iota(jnp.int32, (EBLK, 128), 0)
    sel = jnp.where(lane == pos % 128, vb, 0.0)
    return jnp.sum(sel, axis=1, keepdims=True)


def _wfilt_body(attr_ref, w_ref, we1_ref, be1_ref, we2_ref, be2_ref, out_ref):
    attr = _lanes_to_col(attr_ref[...])
    w = _lanes_to_col(w_ref[...])
    step = CUTOFF / (NG - 1)
    offset = lax.broadcasted_iota(jnp.int32, (1, NG), 1).astype(jnp.float32) * step
    coeff = -0.5 / step ** 2
    dist = attr - offset
    smeared = jnp.exp(coeff * dist * dist)
    hgt = jnp.dot(smeared, we1_ref[...], preferred_element_type=jnp.float32)
    hgt = jnp.maximum(hgt + be1_ref[...], 0.0)
    wf = jnp.dot(hgt, we2_ref[...], preferred_element_type=jnp.float32)
    wf = wf + be2_ref[...]
    cc = 0.5 * (jnp.cos(w * (math.pi / CUTOFF)) + 1.0)
    out_ref[...] = wf * cc


_wfilt_call = pl.pallas_call(
    _wfilt_body,
    grid=(EGRID,),
    in_specs=[
        pl.BlockSpec((1, ERB, 128), lambda i: (i, 0, 0)),
        pl.BlockSpec((1, ERB, 128), lambda i: (i, 0, 0)),
        pl.BlockSpec((NG, NF), lambda i: (0, 0)),
        pl.BlockSpec((1, NF), lambda i: (0, 0)),
        pl.BlockSpec((NF, NFP), lambda i: (0, 0)),
        pl.BlockSpec((1, NFP), lambda i: (0, 0)),
    ],
    out_specs=pl.BlockSpec((EBLK, NFP), lambda i: (i, 0)),
    out_shape=jax.ShapeDtypeStruct((E, NFP), jnp.float32),
)


def _h_body(x_ref, w_ref, out_ref):
    out_ref[...] = jnp.dot(x_ref[...], w_ref[...],
                           preferred_element_type=jnp.float32)


_h_call = pl.pallas_call(
    _h_body,
    out_shape=jax.ShapeDtypeStruct((N, NFP), jnp.float32),
)


def _sc_body(h_hbm, wf_hbm, src_hbm, dst_hbm, out_hbm,
             agg_sh, src_v, dst_v, wf_v, rows_v, zero_v, sem):
    cid = lax.axis_index("c")
    sid = lax.axis_index("s")
    wid = sid * NC + cid

    @pl.when(sid < ZTILES)
    def _zero():
        def _zrow(i, carry):
            for j in range(NFP // L):
                zero_v[i, pl.ds(j * L, L)] = jnp.zeros((L,), jnp.float32)
            return carry
        lax.fori_loop(0, ZROWS, _zrow, 0)

        def _zcopy(k, carry):
            zbase = pl.multiple_of(sid * ZPT + k * ZROWS, 8)
            pltpu.sync_copy(zero_v, agg_sh.at[pl.ds(zbase, ZROWS)])
            return carry
        lax.fori_loop(0, ZREP, _zcopy, 0)
    plsc.subcore_barrier()

    def _chunk(c, carry):
        base = pl.multiple_of(wid * EPW + c * CHUNK, 8)
        pltpu.sync_copy(src_hbm.at[pl.ds(base, CHUNK)], src_v)
        pltpu.sync_copy(dst_hbm.at[pl.ds(base, CHUNK)], dst_v)
        pltpu.sync_copy(wf_hbm.at[pl.ds(base, CHUNK)], wf_v)
        pltpu.async_copy(h_hbm.at[src_v], rows_v, sem).wait()

        def _mul(i, icarry):
            for j in range(NF // L):
                s = pl.ds(j * L, L)
                rows_v[i, s] = rows_v[i, s] * wf_v[i, s]
            return icarry
        lax.fori_loop(0, CHUNK, _mul, 0)

        pltpu.sync_copy(rows_v, agg_sh.at[dst_v], add=True)
        return carry
    lax.fori_loop(0, NCHUNK, _chunk, 0)

    plsc.subcore_barrier()

    @pl.when(sid < ZTILES)
    def _copy_out():
        def _ocopy(k, carry):
            zbase = pl.multiple_of(sid * ZPT + k * ZROWS, 8)
            obase = pl.multiple_of(cid * N + sid * ZPT + k * ZROWS, 8)
            pltpu.sync_copy(agg_sh.at[pl.ds(zbase, ZROWS)],
                            out_hbm.at[pl.ds(obase, ZROWS)])
            return carry
        lax.fori_loop(0, ZREP, _ocopy, 0)


_sc_call = functools.partial(
    pl.kernel,
    mesh=plsc.VectorSubcoreMesh(core_axis_name="c", subcore_axis_name="s"),
    out_type=jax.ShapeDtypeStruct((NC * N, NFP), jnp.float32),
    scratch_types=[
        pltpu.VMEM_SHARED((N, NFP), jnp.float32),
        pltpu.VMEM((CHUNK,), jnp.int32),
        pltpu.VMEM((CHUNK,), jnp.int32),
        pltpu.VMEM((CHUNK, NFP), jnp.float32),
        pltpu.VMEM((CHUNK, NFP), jnp.float32),
        pltpu.VMEM((ZROWS, NFP), jnp.float32),
        pltpu.SemaphoreType.DMA,
    ],
)(_sc_body)


def _final_body(agg_ref, x_ref, w_ref, b_ref, out_ref):
    aggsum = agg_ref[:N, :NF] + agg_ref[N:, :NF]
    y = jnp.dot(aggsum, w_ref[...], preferred_element_type=jnp.float32)
    y = jnp.maximum(y + b_ref[...], 0.0)
    out_ref[...] = x_ref[...] + y


_final_call = pl.pallas_call(
    _final_body,
    out_shape=jax.ShapeDtypeStruct((N, D), jnp.float32),
)


def kernel(x, edge_index, edge_attr, edge_weight, W_lin1, W_e1, b_e1,
           W_e2, b_e2, W_lin2, b_lin2):
    src = edge_index[0].astype(jnp.int32)
    dst = edge_index[1].astype(jnp.int32)
    pad = jnp.zeros((NF, NFP - NF), jnp.float32)
    we2p = jnp.concatenate([W_e2, pad], axis=1)
    be2p = jnp.concatenate([b_e2, jnp.zeros((NFP - NF,), jnp.float32)])
    w1p = jnp.concatenate([W_lin1, jnp.zeros((D, NFP - NF), jnp.float32)],
                          axis=1)
    wfilt = _wfilt_call(edge_attr.reshape(EGRID, ERB, 128),
                        edge_weight.reshape(EGRID, ERB, 128),
                        W_e1, b_e1.reshape(1, NF),
                        we2p, be2p.reshape(1, NFP))
    h = _h_call(x, w1p)
    agg2 = _sc_call(h, wfilt, src, dst)
    return _final_call(agg2, x, W_lin2, b_lin2.reshape(1, D))


# speedup vs baseline: 1.8726x; 1.2285x over previous
import functools
import math

import jax
import jax.numpy as jnp
from jax import lax
from jax.experimental import pallas as pl
from jax.experimental.pallas import tpu as pltpu
from jax.experimental.pallas import tpu_sc as plsc

N = 10000
E = 320000
EH = E // 2
D = 128
NG = 50
NF = 64
NFP = 128
CUTOFF = 5.0

NC = 2
NS = 16
L = 16
NW = NC * NS
RPW = EH // NW
CHUNK = 80
CR = CHUNK // 2
NCHUNK = RPW // CR
NBUF = 3
ZTILES = 10
ZPT = N // ZTILES
ZROWS = 40
ZREP = ZPT // ZROWS

ERB = 25
EBLK = ERB * 128
EGRID2 = EH // EBLK


def _lanes_to_col(v):
    vb = jnp.broadcast_to(v.reshape(ERB, 1, 128), (ERB, 128, 128))
    vb = vb.reshape(EBLK, 128)
    lane = lax.broadcasted_iota(jnp.int32, (EBLK, 128), 1)
    pos = lax.broadcasted_iota(jnp.int32, (EBLK, 128), 0)
    sel = jnp.where(lane == pos % 128, vb, 0.0)
    return jnp.sum(sel, axis=1, keepdims=True)


def _wfilt_body(a1_ref, a2_ref, w1_ref, w2_ref, we1_ref, be1_ref, we2_ref,
                be2_ref, out_ref):
    step = CUTOFF / (NG - 1)
    coeff = -0.5 / step ** 2
    offset = lax.broadcasted_iota(jnp.int32, (1, NG), 1).astype(jnp.float32) * step

    def half(attr_raw, w_raw):
        attr = _lanes_to_col(attr_raw)
        w = _lanes_to_col(w_raw)
        dist = attr - offset
        smeared = jnp.exp(coeff * dist * dist)
        hgt = jnp.dot(smeared, we1_ref[...],
                      preferred_element_type=jnp.float32)
        hgt = jnp.maximum(hgt + be1_ref[...], 0.0)
        wf = jnp.dot(hgt, we2_ref[...],
                     preferred_element_type=jnp.float32)
        wf = wf + be2_ref[...]
        cc = 0.5 * (jnp.cos(w * (math.pi / CUTOFF)) + 1.0)
        return wf * cc

    wfa = half(a1_ref[...], w1_ref[...])
    wfb = half(a2_ref[...], w2_ref[...])
    out_ref[...] = jnp.concatenate([wfa, wfb], axis=1)


_wfilt_call = pl.pallas_call(
    _wfilt_body,
    grid=(EGRID2,),
    in_specs=[
        pl.BlockSpec((1, ERB, 128), lambda i: (i, 0, 0)),
        pl.BlockSpec((1, ERB, 128), lambda i: (i + EGRID2, 0, 0)),
        pl.BlockSpec((1, ERB, 128), lambda i: (i, 0, 0)),
        pl.BlockSpec((1, ERB, 128), lambda i: (i + EGRID2, 0, 0)),
        pl.BlockSpec((NG, NF), lambda i: (0, 0)),
        pl.BlockSpec((1, NF), lambda i: (0, 0)),
        pl.BlockSpec((NF, NF), lambda i: (0, 0)),
        pl.BlockSpec((1, NF), lambda i: (0, 0)),
    ],
    out_specs=pl.BlockSpec((EBLK, NFP), lambda i: (i, 0)),
    out_shape=jax.ShapeDtypeStruct((EH, NFP), jnp.float32),
)


def _h_body(x_ref, w_ref, out_ref):
    out_ref[...] = jnp.dot(x_ref[...], w_ref[...],
                           preferred_element_type=jnp.float32)


_h_call = pl.pallas_call(
    _h_body,
    out_shape=jax.ShapeDtypeStruct((N, NFP), jnp.float32),
)


def _sc_body(h_hbm, wf_hbm, src_hbm, dst_hbm, out_hbm,
             agg_sh, src_v, dst_v, wf_v, rows_v, sem_in, sem_g,
             sem_s):
    cid = lax.axis_index("c")
    sid = lax.axis_index("s")
    wid = sid * NC + cid
    rb0 = wid * RPW

    @pl.when(sid < ZTILES)
    def _zero():
        def _zrow(i, carry):
            for j in range(NFP // L):
                rows_v[0, i, pl.ds(j * L, L)] = jnp.zeros((L,), jnp.float32)
            return carry
        lax.fori_loop(0, ZROWS, _zrow, 0)

        def _zcopy(k, carry):
            zbase = pl.multiple_of(sid * ZPT + k * ZROWS, 8)
            pltpu.sync_copy(rows_v.at[0, pl.ds(0, ZROWS)],
                            agg_sh.at[pl.ds(zbase, ZROWS)])
            return carry
        lax.fori_loop(0, ZREP, _zcopy, 0)
    plsc.subcore_barrier()

    def _front_copies(c):
        b = c % NBUF
        rbase = pl.multiple_of(rb0 + c * CR, 8)
        return (
            (src_hbm.at[pl.ds(rbase, CR)], src_v.at[b, pl.ds(0, CR)]),
            (src_hbm.at[pl.ds(EH + rbase, CR)], src_v.at[b, pl.ds(CR, CR)]),
            (dst_hbm.at[pl.ds(rbase, CR)], dst_v.at[b, pl.ds(0, CR)]),
            (dst_hbm.at[pl.ds(EH + rbase, CR)], dst_v.at[b, pl.ds(CR, CR)]),
            (wf_hbm.at[pl.ds(rbase, CR)], wf_v.at[b]),
        )

    def _front(c):
        for s, d in _front_copies(c):
            pltpu.async_copy(s, d, sem_in)

    def _front_wait(c):
        for s, d in _front_copies(c):
            pltpu.make_async_copy(s, d, sem_in).wait()

    def _gather(c):
        b = c % NBUF
        pltpu.async_copy(h_hbm.at[src_v.at[b]], rows_v.at[b], sem_g)

    def _gather_wait(c):
        b = c % NBUF
        pltpu.make_async_copy(h_hbm.at[src_v.at[b]], rows_v.at[b],
                              sem_g).wait()

    def _scatter(c):
        b = c % NBUF
        pltpu.async_copy(rows_v.at[b], agg_sh.at[dst_v.at[b]], sem_s,
                         add=True)

    def _scatter_wait(c):
        b = c % NBUF
        pltpu.make_async_copy(rows_v.at[b], agg_sh.at[dst_v.at[b]],
                              sem_s).wait()

    _front(0)
    _front(1)
    _front_wait(0)
    _gather(0)

    def _step(c, carry):
        b = c % NBUF
        _gather_wait(c)

        @pl.when(c + 1 < NCHUNK)
        def _():
            _front_wait(c + 1)

            @pl.when(c >= 2)
            def _():
                _scatter_wait(c - 2)
            _gather(c + 1)

        @pl.when(c + 2 < NCHUNK)
        def _():
            _front(c + 2)

        def _mul(i, icarry):
            for j in range(NF // L):
                s = pl.ds(j * L, L)
                s2 = pl.ds(NF + j * L, L)
                rows_v[b, i, s] = rows_v[b, i, s] * wf_v[b, i, s]
                rows_v[b, CR + i, s] = rows_v[b, CR + i, s] * wf_v[b, i, s2]
            return icarry
        lax.fori_loop(0, CR, _mul, 0)

        _scatter(c)
        return carry
    lax.fori_loop(0, NCHUNK, _step, 0)
    _scatter_wait(NCHUNK - 2)
    _scatter_wait(NCHUNK - 1)

    plsc.subcore_barrier()

    @pl.when(sid < ZTILES)
    def _copy_out():
        def _ocopy(k, carry):
            zbase = pl.multiple_of(sid * ZPT + k * ZROWS, 8)
            obase = pl.multiple_of(cid * N + sid * ZPT + k * ZROWS, 8)
            pltpu.sync_copy(agg_sh.at[pl.ds(zbase, ZROWS)],
                            out_hbm.at[pl.ds(obase, ZROWS)])
            return carry
        lax.fori_loop(0, ZREP, _ocopy, 0)


_sc_call = functools.partial(
    pl.kernel,
    mesh=plsc.VectorSubcoreMesh(core_axis_name="c", subcore_axis_name="s"),
    out_type=jax.ShapeDtypeStruct((NC * N, NFP), jnp.float32),
    scratch_types=[
        pltpu.VMEM_SHARED((N, NFP), jnp.float32),
        pltpu.VMEM((NBUF, CHUNK), jnp.int32),
        pltpu.VMEM((NBUF, CHUNK), jnp.int32),
        pltpu.VMEM((NBUF, CR, NFP), jnp.float32),
        pltpu.VMEM((NBUF, CHUNK, NFP), jnp.float32),
        pltpu.SemaphoreType.DMA,
        pltpu.SemaphoreType.DMA,
        pltpu.SemaphoreType.DMA,
    ],
)(_sc_body)


def _final_body(agg_ref, x_ref, w_ref, b_ref, out_ref):
    aggsum = agg_ref[:N, :NF] + agg_ref[N:, :NF]
    y = jnp.dot(aggsum, w_ref[...], preferred_element_type=jnp.float32)
    y = jnp.maximum(y + b_ref[...], 0.0)
    out_ref[...] = x_ref[...] + y


_final_call = pl.pallas_call(
    _final_body,
    out_shape=jax.ShapeDtypeStruct((N, D), jnp.float32),
)


def kernel(x, edge_index, edge_attr, edge_weight, W_lin1, W_e1, b_e1,
           W_e2, b_e2, W_lin2, b_lin2):
    src = edge_index[0].astype(jnp.int32)
    dst = edge_index[1].astype(jnp.int32)
    attr3 = edge_attr.reshape(2 * EGRID2, ERB, 128)
    wgt3 = edge_weight.reshape(2 * EGRID2, ERB, 128)
    w1p = jnp.concatenate([W_lin1, jnp.zeros((D, NFP - NF), jnp.float32)],
                          axis=1)
    wfilt = _wfilt_call(attr3, attr3, wgt3, wgt3,
                        W_e1, b_e1.reshape(1, NF),
                        W_e2, b_e2.reshape(1, NF))
    h = _h_call(x, w1p)
    agg2 = _sc_call(h, wfilt, src, dst)
    return _final_call(agg2, x, W_lin2, b_lin2.reshape(1, D))


# speedup vs baseline: 4.3948x; 2.3469x over previous
import functools
import math

import jax
import jax.numpy as jnp
from jax import lax
from jax.experimental import pallas as pl
from jax.experimental.pallas import tpu as pltpu
from jax.experimental.pallas import tpu_sc as plsc

N = 10000
E = 320000
EH = E // 2
D = 128
NG = 50
NF = 64
NFP = 128
CUTOFF = 5.0

NC = 2
NS = 16
L = 16
NW = NC * NS
RPW = EH // NW
CHUNK = 80
CR = CHUNK // 2
NCHUNK = RPW // CR
NBUF = 3
ZTILES = 10
ZPT = N // ZTILES
ZROWS = 40
ZREP = ZPT // ZROWS

ERB = 25
EBLK = ERB * 128
EGRID2 = EH // EBLK


def _wfilt_body(a1_ref, a2_ref, w1_ref, w2_ref, we1t_ref, be1_ref,
                we2a_ref, out_ref):
    step = CUTOFF / (NG - 1)
    coeff = -0.5 / step ** 2
    offs = lax.broadcasted_iota(jnp.int32, (NG, 1), 0).astype(jnp.float32) * step

    def half(attr_row, w_row):
        attr_row = attr_row.reshape(1, EBLK)
        w_row = w_row.reshape(1, EBLK)
        dist = attr_row - offs
        smeared = jnp.exp(coeff * dist * dist)
        hgt = lax.dot_general(we1t_ref[...], smeared,
                              (((1,), (0,)), ((), ())),
                              preferred_element_type=jnp.float32)
        hgt = jnp.maximum(hgt + be1_ref[...], 0.0)
        cc = 0.5 * (jnp.cos(w_row * (math.pi / CUTOFF)) + 1.0)
        aug = jnp.concatenate([hgt * cc, cc], axis=0)
        return lax.dot_general(aug, we2a_ref[...],
                               (((0,), (0,)), ((), ())),
                               preferred_element_type=jnp.float32)

    wfa = half(a1_ref[...], w1_ref[...])
    wfb = half(a2_ref[...], w2_ref[...])
    out_ref[...] = jnp.concatenate([wfa, wfb], axis=1)


_wfilt_call = pl.pallas_call(
    _wfilt_body,
    grid=(EGRID2,),
    in_specs=[
        pl.BlockSpec((1, 1, EBLK), lambda i: (i, 0, 0)),
        pl.BlockSpec((1, 1, EBLK), lambda i: (i + EGRID2, 0, 0)),
        pl.BlockSpec((1, 1, EBLK), lambda i: (i, 0, 0)),
        pl.BlockSpec((1, 1, EBLK), lambda i: (i + EGRID2, 0, 0)),
        pl.BlockSpec((NF, NG), lambda i: (0, 0)),
        pl.BlockSpec((NF, 1), lambda i: (0, 0)),
        pl.BlockSpec((NF + 1, NF), lambda i: (0, 0)),
    ],
    out_specs=pl.BlockSpec((EBLK, NFP), lambda i: (i, 0)),
    out_shape=jax.ShapeDtypeStruct((EH, NFP), jnp.float32),
)


def _h_body(x_ref, w_ref, out_ref):
    out_ref[...] = jnp.dot(x_ref[...], w_ref[...],
                           preferred_element_type=jnp.float32)


_h_call = pl.pallas_call(
    _h_body,
    out_shape=jax.ShapeDtypeStruct((N, NFP), jnp.float32),
)


def _sc_body(h_hbm, wf_hbm, src_hbm, dst_hbm, out_hbm,
             agg_sh, src_v, dst_v, wf_v, rows_v, sem_in, sem_g,
             sem_s):
    cid = lax.axis_index("c")
    sid = lax.axis_index("s")
    wid = sid * NC + cid
    rb0 = wid * RPW

    @pl.when(sid < ZTILES)
    def _zero():
        def _zrow(i, carry):
            for j in range(NFP // L):
                rows_v[0, i, pl.ds(j * L, L)] = jnp.zeros((L,), jnp.float32)
            return carry
        lax.fori_loop(0, ZROWS, _zrow, 0)

        def _zcopy(k, carry):
            zbase = pl.multiple_of(sid * ZPT + k * ZROWS, 8)
            pltpu.sync_copy(rows_v.at[0, pl.ds(0, ZROWS)],
                            agg_sh.at[pl.ds(zbase, ZROWS)])
            return carry
        lax.fori_loop(0, ZREP, _zcopy, 0)
    plsc.subcore_barrier()

    def _front_copies(c):
        b = c % NBUF
        rbase = pl.multiple_of(rb0 + c * CR, 8)
        return (
            (src_hbm.at[pl.ds(rbase, CR)], src_v.at[b, pl.ds(0, CR)]),
            (src_hbm.at[pl.ds(EH + rbase, CR)], src_v.at[b, pl.ds(CR, CR)]),
            (dst_hbm.at[pl.ds(rbase, CR)], dst_v.at[b, pl.ds(0, CR)]),
            (dst_hbm.at[pl.ds(EH + rbase, CR)], dst_v.at[b, pl.ds(CR, CR)]),
            (wf_hbm.at[pl.ds(rbase, CR)], wf_v.at[b]),
        )

    def _front(c):
        for s, d in _front_copies(c):
            pltpu.async_copy(s, d, sem_in)

    def _front_wait(c):
        for s, d in _front_copies(c):
            pltpu.make_async_copy(s, d, sem_in).wait()

    def _gather(c):
        b = c % NBUF
        pltpu.async_copy(h_hbm.at[src_v.at[b]], rows_v.at[b], sem_g)

    def _gather_wait(c):
        b = c % NBUF
        pltpu.make_async_copy(h_hbm.at[src_v.at[b]], rows_v.at[b],
                              sem_g).wait()

    def _scatter(c):
        b = c % NBUF
        pltpu.async_copy(rows_v.at[b], agg_sh.at[dst_v.at[b]], sem_s,
                         add=True)

    def _scatter_wait(c):
        b = c % NBUF
        pltpu.make_async_copy(rows_v.at[b], agg_sh.at[dst_v.at[b]],
                              sem_s).wait()

    _front(0)
    _front(1)
    _front_wait(0)
    _gather(0)

    def _step(c, carry):
        b = c % NBUF
        _gather_wait(c)

        @pl.when(c + 1 < NCHUNK)
        def _():
            _front_wait(c + 1)

            @pl.when(c >= 2)
            def _():
                _scatter_wait(c - 2)
            _gather(c + 1)

        @pl.when(c + 2 < NCHUNK)
        def _():
            _front(c + 2)

        def _mul(i, icarry):
            for j in range(NF // L):
                s = pl.ds(j * L, L)
                s2 = pl.ds(NF + j * L, L)
                rows_v[b, i, s] = rows_v[b, i, s] * wf_v[b, i, s]
                rows_v[b, CR + i, s] = rows_v[b, CR + i, s] * wf_v[b, i, s2]
            return icarry
        lax.fori_loop(0, CR, _mul, 0)

        _scatter(c)
        return carry
    lax.fori_loop(0, NCHUNK, _step, 0)
    _scatter_wait(NCHUNK - 2)
    _scatter_wait(NCHUNK - 1)

    plsc.subcore_barrier()

    @pl.when(sid < ZTILES)
    def _copy_out():
        def _ocopy(k, carry):
            zbase = pl.multiple_of(sid * ZPT + k * ZROWS, 8)
            obase = pl.multiple_of(cid * N + sid * ZPT + k * ZROWS, 8)
            pltpu.sync_copy(agg_sh.at[pl.ds(zbase, ZROWS)],
                            out_hbm.at[pl.ds(obase, ZROWS)])
            return carry
        lax.fori_loop(0, ZREP, _ocopy, 0)


_sc_call = functools.partial(
    pl.kernel,
    mesh=plsc.VectorSubcoreMesh(core_axis_name="c", subcore_axis_name="s"),
    out_type=jax.ShapeDtypeStruct((NC * N, NFP), jnp.float32),
    scratch_types=[
        pltpu.VMEM_SHARED((N, NFP), jnp.float32),
        pltpu.VMEM((NBUF, CHUNK), jnp.int32),
        pltpu.VMEM((NBUF, CHUNK), jnp.int32),
        pltpu.VMEM((NBUF, CR, NFP), jnp.float32),
        pltpu.VMEM((NBUF, CHUNK, NFP), jnp.float32),
        pltpu.SemaphoreType.DMA,
        pltpu.SemaphoreType.DMA,
        pltpu.SemaphoreType.DMA,
    ],
)(_sc_body)


def _final_body(agg_ref, x_ref, w_ref, b_ref, out_ref):
    aggsum = agg_ref[:N, :NF] + agg_ref[N:, :NF]
    y = jnp.dot(aggsum, w_ref[...], preferred_element_type=jnp.float32)
    y = jnp.maximum(y + b_ref[...], 0.0)
    out_ref[...] = x_ref[...] + y


_final_call = pl.pallas_call(
    _final_body,
    out_shape=jax.ShapeDtypeStruct((N, D), jnp.float32),
)


def kernel(x, edge_index, edge_attr, edge_weight, W_lin1, W_e1, b_e1,
           W_e2, b_e2, W_lin2, b_lin2):
    src = edge_index[0].astype(jnp.int32)
    dst = edge_index[1].astype(jnp.int32)
    attr2 = edge_attr.reshape(2 * EGRID2, 1, EBLK)
    wgt2 = edge_weight.reshape(2 * EGRID2, 1, EBLK)
    w1p = jnp.concatenate([W_lin1, jnp.zeros((D, NFP - NF), jnp.float32)],
                          axis=1)
    we2a = jnp.concatenate([W_e2, b_e2.reshape(1, NF)], axis=0)
    wfilt = _wfilt_call(attr2, attr2, wgt2, wgt2,
                        W_e1.T, b_e1.reshape(NF, 1), we2a)
    h = _h_call(x, w1p)
    agg2 = _sc_call(h, wfilt, src, dst)
    return _final_call(agg2, x, W_lin2, b_lin2.reshape(1, D))


# speedup vs baseline: 4.6169x; 1.0505x over previous
import functools
import math

import jax
import jax.numpy as jnp
from jax import lax
from jax.experimental import pallas as pl
from jax.experimental.pallas import tpu as pltpu
from jax.experimental.pallas import tpu_sc as plsc

N = 10000
E = 320000
EH = E // 2
D = 128
NG = 50
NF = 64
NFP = 128
CUTOFF = 5.0

NC = 2
NS = 16
L = 16
NW = NC * NS
RPW = EH // NW
CHUNK = 80
CR = CHUNK // 2
NCHUNK = RPW // CR
NBUF = 3
ZPT = 640
ZLAST = N - 15 * ZPT
ZROWS = 80

ERB = 25
EBLK = ERB * 128
EGRID2 = EH // EBLK


def _wfilt_body(a1_ref, a2_ref, w1_ref, w2_ref, we1t_ref, be1_ref,
                we2a_ref, out_ref):
    step = CUTOFF / (NG - 1)
    coeff = -0.5 / step ** 2
    offs = lax.broadcasted_iota(jnp.int32, (NG, 1), 0).astype(jnp.float32) * step

    def half(attr_row, w_row):
        attr_row = attr_row.reshape(1, EBLK)
        w_row = w_row.reshape(1, EBLK)
        dist = attr_row - offs
        smeared = jnp.exp(coeff * dist * dist)
        hgt = lax.dot_general(we1t_ref[...], smeared,
                              (((1,), (0,)), ((), ())),
                              preferred_element_type=jnp.float32)
        hgt = jnp.maximum(hgt + be1_ref[...], 0.0)
        cc = 0.5 * (jnp.cos(w_row * (math.pi / CUTOFF)) + 1.0)
        aug = jnp.concatenate([hgt * cc, cc], axis=0)
        return lax.dot_general(aug, we2a_ref[...],
                               (((0,), (0,)), ((), ())),
                               preferred_element_type=jnp.float32)

    wfa = half(a1_ref[...], w1_ref[...])
    wfb = half(a2_ref[...], w2_ref[...])
    out_ref[...] = jnp.concatenate([wfa, wfb], axis=1)


_wfilt_call = pl.pallas_call(
    _wfilt_body,
    grid=(EGRID2,),
    in_specs=[
        pl.BlockSpec((1, 1, EBLK), lambda i: (i, 0, 0)),
        pl.BlockSpec((1, 1, EBLK), lambda i: (i + EGRID2, 0, 0)),
        pl.BlockSpec((1, 1, EBLK), lambda i: (i, 0, 0)),
        pl.BlockSpec((1, 1, EBLK), lambda i: (i + EGRID2, 0, 0)),
        pl.BlockSpec((NF, NG), lambda i: (0, 0)),
        pl.BlockSpec((NF, 1), lambda i: (0, 0)),
        pl.BlockSpec((NF + 1, NF), lambda i: (0, 0)),
    ],
    out_specs=pl.BlockSpec((EBLK, NFP), lambda i: (i, 0)),
    out_shape=jax.ShapeDtypeStruct((EH, NFP), jnp.float32),
)


def _h_body(x_ref, w_ref, out_ref):
    out_ref[...] = jnp.dot(x_ref[...], w_ref[...],
                           preferred_element_type=jnp.float32)


_h_call = pl.pallas_call(
    _h_body,
    out_shape=jax.ShapeDtypeStruct((N, NFP), jnp.float32),
)


def _sc_body(h_hbm, wf_hbm, src_hbm, dst_hbm, out_hbm,
             agg_sh, src_v, dst_v, wf_v, rows_v, sem_in, sem_g,
             sem_s):
    cid = lax.axis_index("c")
    sid = lax.axis_index("s")
    wid = sid * NC + cid
    rb0 = wid * RPW

    def _zrow(i, carry):
        for j in range(NFP // L):
            rows_v[0, i, pl.ds(j * L, L)] = jnp.zeros((L,), jnp.float32)
        return carry
    lax.fori_loop(0, ZROWS, _zrow, 0)
    nrep = jnp.where(sid < NS - 1, ZPT // ZROWS, ZLAST // ZROWS)

    def _zcopy(k, carry):
        zbase = pl.multiple_of(sid * ZPT + k * ZROWS, 8)
        pltpu.async_copy(rows_v.at[0], agg_sh.at[pl.ds(zbase, ZROWS)],
                         sem_in)
        return carry
    lax.fori_loop(0, nrep, _zcopy, 0)

    def _zwait(k, carry):
        zbase = pl.multiple_of(sid * ZPT + k * ZROWS, 8)
        pltpu.make_async_copy(rows_v.at[0],
                              agg_sh.at[pl.ds(zbase, ZROWS)],
                              sem_in).wait()
        return carry
    lax.fori_loop(0, nrep, _zwait, 0)
    plsc.subcore_barrier()

    def _front_copies(c):
        b = c % NBUF
        rbase = pl.multiple_of(rb0 + c * CR, 8)
        return (
            (src_hbm.at[pl.ds(rbase, CR)], src_v.at[b, pl.ds(0, CR)]),
            (src_hbm.at[pl.ds(EH + rbase, CR)], src_v.at[b, pl.ds(CR, CR)]),
            (dst_hbm.at[pl.ds(rbase, CR)], dst_v.at[b, pl.ds(0, CR)]),
            (dst_hbm.at[pl.ds(EH + rbase, CR)], dst_v.at[b, pl.ds(CR, CR)]),
            (wf_hbm.at[pl.ds(rbase, CR)], wf_v.at[b]),
        )

    def _front(c):
        for s, d in _front_copies(c):
            pltpu.async_copy(s, d, sem_in)

    def _front_wait(c):
        for s, d in _front_copies(c):
            pltpu.make_async_copy(s, d, sem_in).wait()

    def _gather(c):
        b = c % NBUF
        pltpu.async_copy(h_hbm.at[src_v.at[b]], rows_v.at[b], sem_g)

    def _gather_wait(c):
        b = c % NBUF
        pltpu.make_async_copy(h_hbm.at[src_v.at[b]], rows_v.at[b],
                              sem_g).wait()

    def _scatter(c):
        b = c % NBUF
        pltpu.async_copy(rows_v.at[b], agg_sh.at[dst_v.at[b]], sem_s,
                         add=True)

    def _scatter_wait(c):
        b = c % NBUF
        pltpu.make_async_copy(rows_v.at[b], agg_sh.at[dst_v.at[b]],
                              sem_s).wait()

    _front(0)
    _front(1)
    _front_wait(0)
    _gather(0)

    def _step(c, carry):
        b = c % NBUF
        _gather_wait(c)

        @pl.when(c + 1 < NCHUNK)
        def _():
            _front_wait(c + 1)

            @pl.when(c >= 2)
            def _():
                _scatter_wait(c - 2)
            _gather(c + 1)

        @pl.when(c + 2 < NCHUNK)
        def _():
            _front(c + 2)

        def _mul(i, icarry):
            for j in range(NF // L):
                s = pl.ds(j * L, L)
                s2 = pl.ds(NF + j * L, L)
                rows_v[b, i, s] = rows_v[b, i, s] * wf_v[b, i, s]
                rows_v[b, CR + i, s] = rows_v[b, CR + i, s] * wf_v[b, i, s2]
            return icarry
        lax.fori_loop(0, CR, _mul, 0, unroll=4)

        _scatter(c)
        return carry
    lax.fori_loop(0, NCHUNK, _step, 0)
    _scatter_wait(NCHUNK - 2)
    _scatter_wait(NCHUNK - 1)

    plsc.subcore_barrier()

    zbase = pl.multiple_of(sid * ZPT, 8)
    obase = pl.multiple_of(cid * N + sid * ZPT, 8)

    @pl.when(sid < NS - 1)
    def _copy_out():
        pltpu.sync_copy(agg_sh.at[pl.ds(zbase, ZPT)],
                        out_hbm.at[pl.ds(obase, ZPT)])

    @pl.when(sid == NS - 1)
    def _copy_out_last():
        pltpu.sync_copy(agg_sh.at[pl.ds(zbase, ZLAST)],
                        out_hbm.at[pl.ds(obase, ZLAST)])


_sc_call = functools.partial(
    pl.kernel,
    mesh=plsc.VectorSubcoreMesh(core_axis_name="c", subcore_axis_name="s"),
    out_type=jax.ShapeDtypeStruct((NC * N, NFP), jnp.float32),
    scratch_types=[
        pltpu.VMEM_SHARED((N, NFP), jnp.float32),
        pltpu.VMEM((NBUF, CHUNK), jnp.int32),
        pltpu.VMEM((NBUF, CHUNK), jnp.int32),
        pltpu.VMEM((NBUF, CR, NFP), jnp.float32),
        pltpu.VMEM((NBUF, CHUNK, NFP), jnp.float32),
        pltpu.SemaphoreType.DMA,
        pltpu.SemaphoreType.DMA,
        pltpu.SemaphoreType.DMA,
    ],
)(_sc_body)


def _final_body(agg_ref, x_ref, w_ref, b_ref, out_ref):
    aggsum = agg_ref[:N, :NF] + agg_ref[N:, :NF]
    y = jnp.dot(aggsum, w_ref[...], preferred_element_type=jnp.float32)
    y = jnp.maximum(y + b_ref[...], 0.0)
    out_ref[...] = x_ref[...] + y


_final_call = pl.pallas_call(
    _final_body,
    out_shape=jax.ShapeDtypeStruct((N, D), jnp.float32),
)


def kernel(x, edge_index, edge_attr, edge_weight, W_lin1, W_e1, b_e1,
           W_e2, b_e2, W_lin2, b_lin2):
    src = edge_index[0].astype(jnp.int32)
    dst = edge_index[1].astype(jnp.int32)
    attr2 = edge_attr.reshape(2 * EGRID2, 1, EBLK)
    wgt2 = edge_weight.reshape(2 * EGRID2, 1, EBLK)
    w1p = jnp.concatenate([W_lin1, jnp.zeros((D, NFP - NF), jnp.float32)],
                          axis=1)
    we2a = jnp.concatenate([W_e2, b_e2.reshape(1, NF)], axis=0)
    wfilt = _wfilt_call(attr2, attr2, wgt2, wgt2,
                        W_e1.T, b_e1.reshape(NF, 1), we2a)
    h = _h_call(x, w1p)
    agg2 = _sc_call(h, wfilt, src, dst)
    return _final_call(agg2, x, W_lin2, b_lin2.reshape(1, D))


# speedup vs baseline: 4.6244x; 1.0016x over previous
import functools
import math

import jax
import jax.numpy as jnp
from jax import lax
from jax.experimental import pallas as pl
from jax.experimental.pallas import tpu as pltpu
from jax.experimental.pallas import tpu_sc as plsc

N = 10000
E = 320000
EH = E // 2
D = 128
NG = 50
NF = 64
NFP = 128
CUTOFF = 5.0

NC = 2
NS = 16
L = 16
NW = NC * NS
RPW = EH // NW
CHUNK = 80
CR = CHUNK // 2
NCHUNK = RPW // CR
NBUF = 3
ZPT = 640
ZLAST = N - 15 * ZPT
ZROWS = 80

ERB = 25
EBLK = ERB * 128
EGRID2 = EH // EBLK


def _wfilt_body(a1_ref, a2_ref, w1_ref, w2_ref, we1t_ref, be1_ref,
                we2a_ref, x_ref, wl1_ref, out_ref, h_ref):
    step = CUTOFF / (NG - 1)
    coeff = -0.5 / step ** 2
    offs = lax.broadcasted_iota(jnp.int32, (NG, 1), 0).astype(jnp.float32) * step

    def half(attr_row, w_row):
        attr_row = attr_row.reshape(1, EBLK)
        w_row = w_row.reshape(1, EBLK)
        dist = attr_row - offs
        smeared = jnp.exp(coeff * dist * dist)
        hgt = lax.dot_general(we1t_ref[...], smeared,
                              (((1,), (0,)), ((), ())),
                              preferred_element_type=jnp.float32)
        hgt = jnp.maximum(hgt + be1_ref[...], 0.0)
        cc = 0.5 * (jnp.cos(w_row * (math.pi / CUTOFF)) + 1.0)
        aug = jnp.concatenate([hgt * cc, cc], axis=0)
        return lax.dot_general(aug, we2a_ref[...],
                               (((0,), (0,)), ((), ())),
                               preferred_element_type=jnp.float32)

    wfa = half(a1_ref[...], w1_ref[...])
    wfb = half(a2_ref[...], w2_ref[...])
    out_ref[...] = jnp.concatenate([wfa, wfb], axis=1)

    @pl.when(pl.program_id(0) == 0)
    def _h():
        h_ref[...] = jnp.dot(x_ref[...], wl1_ref[...],
                             preferred_element_type=jnp.float32)


_wfilt_call = pl.pallas_call(
    _wfilt_body,
    grid=(EGRID2,),
    in_specs=[
        pl.BlockSpec((1, 1, EBLK), lambda i: (i, 0, 0)),
        pl.BlockSpec((1, 1, EBLK), lambda i: (i + EGRID2, 0, 0)),
        pl.BlockSpec((1, 1, EBLK), lambda i: (i, 0, 0)),
        pl.BlockSpec((1, 1, EBLK), lambda i: (i + EGRID2, 0, 0)),
        pl.BlockSpec((NF, NG), lambda i: (0, 0)),
        pl.BlockSpec((NF, 1), lambda i: (0, 0)),
        pl.BlockSpec((NF + 1, NF), lambda i: (0, 0)),
        pl.BlockSpec((N, D), lambda i: (0, 0)),
        pl.BlockSpec((D, NFP), lambda i: (0, 0)),
    ],
    out_specs=[
        pl.BlockSpec((EBLK, NFP), lambda i: (i, 0)),
        pl.BlockSpec((N, NFP), lambda i: (0, 0)),
    ],
    out_shape=[
        jax.ShapeDtypeStruct((EH, NFP), jnp.float32),
        jax.ShapeDtypeStruct((N, NFP), jnp.float32),
    ],
)


def _sc_body(h_hbm, wf_hbm, src_hbm, dst_hbm, out_hbm,
             agg_sh, src_v, dst_v, wf_v, rows_v, sem_in, sem_g,
             sem_s):
    cid = lax.axis_index("c")
    sid = lax.axis_index("s")
    wid = sid * NC + cid
    rb0 = wid * RPW

    def _zrow(i, carry):
        for j in range(NFP // L):
            rows_v[0, i, pl.ds(j * L, L)] = jnp.zeros((L,), jnp.float32)
        return carry
    lax.fori_loop(0, ZROWS, _zrow, 0)
    nrep = jnp.where(sid < NS - 1, ZPT // ZROWS, ZLAST // ZROWS)

    def _zcopy(k, carry):
        zbase = pl.multiple_of(sid * ZPT + k * ZROWS, 8)
        pltpu.async_copy(rows_v.at[0], agg_sh.at[pl.ds(zbase, ZROWS)],
                         sem_in)
        return carry
    lax.fori_loop(0, nrep, _zcopy, 0)

    def _zwait(k, carry):
        zbase = pl.multiple_of(sid * ZPT + k * ZROWS, 8)
        pltpu.make_async_copy(rows_v.at[0],
                              agg_sh.at[pl.ds(zbase, ZROWS)],
                              sem_in).wait()
        return carry
    lax.fori_loop(0, nrep, _zwait, 0)
    plsc.subcore_barrier()

    def _front_copies(c):
        b = c % NBUF
        rbase = pl.multiple_of(rb0 + c * CR, 8)
        return (
            (src_hbm.at[pl.ds(rbase, CR)], src_v.at[b, pl.ds(0, CR)]),
            (src_hbm.at[pl.ds(EH + rbase, CR)], src_v.at[b, pl.ds(CR, CR)]),
            (dst_hbm.at[pl.ds(rbase, CR)], dst_v.at[b, pl.ds(0, CR)]),
            (dst_hbm.at[pl.ds(EH + rbase, CR)], dst_v.at[b, pl.ds(CR, CR)]),
            (wf_hbm.at[pl.ds(rbase, CR)], wf_v.at[b]),
        )

    def _front(c):
        for s, d in _front_copies(c):
            pltpu.async_copy(s, d, sem_in)

    def _front_wait(c):
        for s, d in _front_copies(c):
            pltpu.make_async_copy(s, d, sem_in).wait()

    def _gather(c):
        b = c % NBUF
        pltpu.async_copy(h_hbm.at[src_v.at[b]], rows_v.at[b], sem_g)

    def _gather_wait(c):
        b = c % NBUF
        pltpu.make_async_copy(h_hbm.at[src_v.at[b]], rows_v.at[b],
                              sem_g).wait()

    def _scatter(c):
        b = c % NBUF
        pltpu.async_copy(rows_v.at[b], agg_sh.at[dst_v.at[b]], sem_s,
                         add=True)

    def _scatter_wait(c):
        b = c % NBUF
        pltpu.make_async_copy(rows_v.at[b], agg_sh.at[dst_v.at[b]],
                              sem_s).wait()

    _front(0)
    _front(1)
    _front_wait(0)
    _gather(0)

    def _step(c, carry):
        b = c % NBUF
        _gather_wait(c)

        @pl.when(c + 1 < NCHUNK)
        def _():
            _front_wait(c + 1)

            @pl.when(c >= 2)
            def _():
                _scatter_wait(c - 2)
            _gather(c + 1)

        @pl.when(c + 2 < NCHUNK)
        def _():
            _front(c + 2)

        def _mul(i, icarry):
            for j in range(NF // L):
                s = pl.ds(j * L, L)
                s2 = pl.ds(NF + j * L, L)
                rows_v[b, i, s] = rows_v[b, i, s] * wf_v[b, i, s]
                rows_v[b, CR + i, s] = rows_v[b, CR + i, s] * wf_v[b, i, s2]
            return icarry
        lax.fori_loop(0, CR, _mul, 0, unroll=4)

        _scatter(c)
        return carry
    lax.fori_loop(0, NCHUNK, _step, 0)
    _scatter_wait(NCHUNK - 2)
    _scatter_wait(NCHUNK - 1)

    plsc.subcore_barrier()

    zbase = pl.multiple_of(sid * ZPT, 8)
    obase = pl.multiple_of(cid * N + sid * ZPT, 8)

    @pl.when(sid < NS - 1)
    def _copy_out():
        pltpu.sync_copy(agg_sh.at[pl.ds(zbase, ZPT)],
                        out_hbm.at[pl.ds(obase, ZPT)])

    @pl.when(sid == NS - 1)
    def _copy_out_last():
        pltpu.sync_copy(agg_sh.at[pl.ds(zbase, ZLAST)],
                        out_hbm.at[pl.ds(obase, ZLAST)])


_sc_call = functools.partial(
    pl.kernel,
    mesh=plsc.VectorSubcoreMesh(core_axis_name="c", subcore_axis_name="s"),
    out_type=jax.ShapeDtypeStruct((NC * N, NFP), jnp.float32),
    scratch_types=[
        pltpu.VMEM_SHARED((N, NFP), jnp.float32),
        pltpu.VMEM((NBUF, CHUNK), jnp.int32),
        pltpu.VMEM((NBUF, CHUNK), jnp.int32),
        pltpu.VMEM((NBUF, CR, NFP), jnp.float32),
        pltpu.VMEM((NBUF, CHUNK, NFP), jnp.float32),
        pltpu.SemaphoreType.DMA,
        pltpu.SemaphoreType.DMA,
        pltpu.SemaphoreType.DMA,
    ],
)(_sc_body)


def _final_body(agg_ref, x_ref, w_ref, b_ref, out_ref):
    aggsum = agg_ref[:N, :NF] + agg_ref[N:, :NF]
    y = jnp.dot(aggsum, w_ref[...], preferred_element_type=jnp.float32)
    y = jnp.maximum(y + b_ref[...], 0.0)
    out_ref[...] = x_ref[...] + y


_final_call = pl.pallas_call(
    _final_body,
    out_shape=jax.ShapeDtypeStruct((N, D), jnp.float32),
)


def kernel(x, edge_index, edge_attr, edge_weight, W_lin1, W_e1, b_e1,
           W_e2, b_e2, W_lin2, b_lin2):
    src = edge_index[0].astype(jnp.int32)
    dst = edge_index[1].astype(jnp.int32)
    attr2 = edge_attr.reshape(2 * EGRID2, 1, EBLK)
    wgt2 = edge_weight.reshape(2 * EGRID2, 1, EBLK)
    w1p = jnp.concatenate([W_lin1, jnp.zeros((D, NFP - NF), jnp.float32)],
                          axis=1)
    we2a = jnp.concatenate([W_e2, b_e2.reshape(1, NF)], axis=0)
    wfilt, h = _wfilt_call(attr2, attr2, wgt2, wgt2,
                           W_e1.T, b_e1.reshape(NF, 1), we2a, x, w1p)
    agg2 = _sc_call(h, wfilt, src, dst)
    return _final_call(agg2, x, W_lin2, b_lin2.reshape(1, D))


# speedup vs baseline: 4.7042x; 1.0173x over previous
import functools
import math

import jax
import jax.numpy as jnp
from jax import lax
from jax.experimental import pallas as pl
from jax.experimental.pallas import tpu as pltpu
from jax.experimental.pallas import tpu_sc as plsc

N = 10000
E = 320000
EH = E // 2
D = 128
NG = 50
NF = 64
NFP = 128
CUTOFF = 5.0

NC = 2
NS = 16
L = 16
NW = NC * NS
RPW = EH // NW
CHUNK = 80
CR = CHUNK // 2
NCHUNK = RPW // CR
NBUF = 3
ZPT = 640
ZLAST = N - 15 * ZPT
ZROWS = 80

EBLK = 16000
EGRID2 = EH // EBLK


def _wfilt_body(a1_ref, a2_ref, w1_ref, w2_ref, we1t_ref, be1_ref,
                we2a_ref, x_ref, wl1_ref, out_ref, h_ref):
    step = CUTOFF / (NG - 1)
    coeff = -0.5 / step ** 2
    offs = lax.broadcasted_iota(jnp.int32, (NG, 1), 0).astype(jnp.float32) * step

    def half(attr_row, w_row):
        attr_row = attr_row.reshape(1, EBLK)
        w_row = w_row.reshape(1, EBLK)
        dist = attr_row - offs
        smeared = jnp.exp(coeff * dist * dist)
        hgt = lax.dot_general(we1t_ref[...], smeared,
                              (((1,), (0,)), ((), ())),
                              preferred_element_type=jnp.float32)
        hgt = jnp.maximum(hgt + be1_ref[...], 0.0)
        cc = 0.5 * (jnp.cos(w_row * (math.pi / CUTOFF)) + 1.0)
        aug = jnp.concatenate([hgt * cc, cc], axis=0)
        return lax.dot_general(aug, we2a_ref[...],
                               (((0,), (0,)), ((), ())),
                               preferred_element_type=jnp.float32)

    wfa = half(a1_ref[...], w1_ref[...])
    wfb = half(a2_ref[...], w2_ref[...])
    out_ref[...] = jnp.concatenate([wfa, wfb], axis=1)

    @pl.when(pl.program_id(0) == 0)
    def _h():
        h_ref[...] = jnp.dot(x_ref[...], wl1_ref[...],
                             preferred_element_type=jnp.float32)


_wfilt_call = pl.pallas_call(
    _wfilt_body,
    grid=(EGRID2,),
    in_specs=[
        pl.BlockSpec((1, 1, EBLK), lambda i: (i, 0, 0)),
        pl.BlockSpec((1, 1, EBLK), lambda i: (i + EGRID2, 0, 0)),
        pl.BlockSpec((1, 1, EBLK), lambda i: (i, 0, 0)),
        pl.BlockSpec((1, 1, EBLK), lambda i: (i + EGRID2, 0, 0)),
        pl.BlockSpec((NF, NG), lambda i: (0, 0)),
        pl.BlockSpec((NF, 1), lambda i: (0, 0)),
        pl.BlockSpec((NF + 1, NF), lambda i: (0, 0)),
        pl.BlockSpec((N, D), lambda i: (0, 0)),
        pl.BlockSpec((D, NFP), lambda i: (0, 0)),
    ],
    out_specs=[
        pl.BlockSpec((EBLK, NFP), lambda i: (i, 0)),
        pl.BlockSpec((N, NFP), lambda i: (0, 0)),
    ],
    out_shape=[
        jax.ShapeDtypeStruct((EH, NFP), jnp.float32),
        jax.ShapeDtypeStruct((N, NFP), jnp.float32),
    ],
)


def _sc_body(h_hbm, wf_hbm, src_hbm, dst_hbm, out_hbm,
             agg_sh, src_v, dst_v, wf_v, rows_v, sem_in, sem_g,
             sem_s):
    cid = lax.axis_index("c")
    sid = lax.axis_index("s")
    wid = sid * NC + cid
    rb0 = wid * RPW

    def _zrow(i, carry):
        for j in range(NFP // L):
            rows_v[0, i, pl.ds(j * L, L)] = jnp.zeros((L,), jnp.float32)
        return carry
    lax.fori_loop(0, ZROWS, _zrow, 0)
    nrep = jnp.where(sid < NS - 1, ZPT // ZROWS, ZLAST // ZROWS)

    def _zcopy(k, carry):
        zbase = pl.multiple_of(sid * ZPT + k * ZROWS, 8)
        pltpu.async_copy(rows_v.at[0], agg_sh.at[pl.ds(zbase, ZROWS)],
                         sem_in)
        return carry
    lax.fori_loop(0, nrep, _zcopy, 0)

    def _zwait(k, carry):
        zbase = pl.multiple_of(sid * ZPT + k * ZROWS, 8)
        pltpu.make_async_copy(rows_v.at[0],
                              agg_sh.at[pl.ds(zbase, ZROWS)],
                              sem_in).wait()
        return carry
    lax.fori_loop(0, nrep, _zwait, 0)
    plsc.subcore_barrier()

    def _front_copies(c):
        b = c % NBUF
        rbase = pl.multiple_of(rb0 + c * CR, 8)
        return (
            (src_hbm.at[pl.ds(rbase, CR)], src_v.at[b, pl.ds(0, CR)]),
            (src_hbm.at[pl.ds(EH + rbase, CR)], src_v.at[b, pl.ds(CR, CR)]),
            (dst_hbm.at[pl.ds(rbase, CR)], dst_v.at[b, pl.ds(0, CR)]),
            (dst_hbm.at[pl.ds(EH + rbase, CR)], dst_v.at[b, pl.ds(CR, CR)]),
            (wf_hbm.at[pl.ds(rbase, CR)], wf_v.at[b]),
        )

    def _front(c):
        for s, d in _front_copies(c):
            pltpu.async_copy(s, d, sem_in)

    def _front_wait(c):
        for s, d in _front_copies(c):
            pltpu.make_async_copy(s, d, sem_in).wait()

    def _gather(c):
        b = c % NBUF
        pltpu.async_copy(h_hbm.at[src_v.at[b]], rows_v.at[b], sem_g)

    def _gather_wait(c):
        b = c % NBUF
        pltpu.make_async_copy(h_hbm.at[src_v.at[b]], rows_v.at[b],
                              sem_g).wait()

    def _scatter(c):
        b = c % NBUF
        pltpu.async_copy(rows_v.at[b], agg_sh.at[dst_v.at[b]], sem_s,
                         add=True)

    def _scatter_wait(c):
        b = c % NBUF
        pltpu.make_async_copy(rows_v.at[b], agg_sh.at[dst_v.at[b]],
                              sem_s).wait()

    _front(0)
    _front(1)
    _front_wait(0)
    _gather(0)

    def _step(c, carry):
        b = c % NBUF
        _gather_wait(c)

        @pl.when(c + 1 < NCHUNK)
        def _():
            _front_wait(c + 1)

            @pl.when(c >= 2)
            def _():
                _scatter_wait(c - 2)
            _gather(c + 1)

        @pl.when(c + 2 < NCHUNK)
        def _():
            _front(c + 2)

        def _mul(i, icarry):
            for j in range(NF // L):
                s = pl.ds(j * L, L)
                s2 = pl.ds(NF + j * L, L)
                rows_v[b, i, s] = rows_v[b, i, s] * wf_v[b, i, s]
                rows_v[b, CR + i, s] = rows_v[b, CR + i, s] * wf_v[b, i, s2]
            return icarry
        lax.fori_loop(0, CR, _mul, 0, unroll=4)

        _scatter(c)
        return carry
    lax.fori_loop(0, NCHUNK, _step, 0)
    _scatter_wait(NCHUNK - 2)
    _scatter_wait(NCHUNK - 1)

    plsc.subcore_barrier()

    zbase = pl.multiple_of(sid * ZPT, 8)
    obase = pl.multiple_of(cid * N + sid * ZPT, 8)

    @pl.when(sid < NS - 1)
    def _copy_out():
        pltpu.sync_copy(agg_sh.at[pl.ds(zbase, ZPT)],
                        out_hbm.at[pl.ds(obase, ZPT)])

    @pl.when(sid == NS - 1)
    def _copy_out_last():
        pltpu.sync_copy(agg_sh.at[pl.ds(zbase, ZLAST)],
                        out_hbm.at[pl.ds(obase, ZLAST)])


_sc_call = functools.partial(
    pl.kernel,
    mesh=plsc.VectorSubcoreMesh(core_axis_name="c", subcore_axis_name="s"),
    out_type=jax.ShapeDtypeStruct((NC * N, NFP), jnp.float32),
    scratch_types=[
        pltpu.VMEM_SHARED((N, NFP), jnp.float32),
        pltpu.VMEM((NBUF, CHUNK), jnp.int32),
        pltpu.VMEM((NBUF, CHUNK), jnp.int32),
        pltpu.VMEM((NBUF, CR, NFP), jnp.float32),
        pltpu.VMEM((NBUF, CHUNK, NFP), jnp.float32),
        pltpu.SemaphoreType.DMA,
        pltpu.SemaphoreType.DMA,
        pltpu.SemaphoreType.DMA,
    ],
)(_sc_body)


def _final_body(agg_ref, x_ref, w_ref, b_ref, out_ref):
    aggsum = agg_ref[:N, :NF] + agg_ref[N:, :NF]
    y = jnp.dot(aggsum, w_ref[...], preferred_element_type=jnp.float32)
    y = jnp.maximum(y + b_ref[...], 0.0)
    out_ref[...] = x_ref[...] + y


_final_call = pl.pallas_call(
    _final_body,
    out_shape=jax.ShapeDtypeStruct((N, D), jnp.float32),
)


def kernel(x, edge_index, edge_attr, edge_weight, W_lin1, W_e1, b_e1,
           W_e2, b_e2, W_lin2, b_lin2):
    src = edge_index[0].astype(jnp.int32)
    dst = edge_index[1].astype(jnp.int32)
    attr2 = edge_attr.reshape(2 * EGRID2, 1, EBLK)
    wgt2 = edge_weight.reshape(2 * EGRID2, 1, EBLK)
    w1p = jnp.concatenate([W_lin1, jnp.zeros((D, NFP - NF), jnp.float32)],
                          axis=1)
    we2a = jnp.concatenate([W_e2, b_e2.reshape(1, NF)], axis=0)
    wfilt, h = _wfilt_call(attr2, attr2, wgt2, wgt2,
                           W_e1.T, b_e1.reshape(NF, 1), we2a, x, w1p)
    agg2 = _sc_call(h, wfilt, src, dst)
    return _final_call(agg2, x, W_lin2, b_lin2.reshape(1, D))


# speedup vs baseline: 4.8252x; 1.0257x over previous
import functools
import math

import jax
import jax.numpy as jnp
from jax import lax
from jax.experimental import pallas as pl
from jax.experimental.pallas import tpu as pltpu
from jax.experimental.pallas import tpu_sc as plsc

N = 10000
E = 320000
EH = E // 2
D = 128
NG = 50
NGA = 56
NF = 64
NFP = 128
CUTOFF = 5.0

NC = 2
NS = 16
L = 16
NW = NC * NS
RPW = EH // NW
CHUNK = 80
CR = CHUNK // 2
NCHUNK = RPW // CR
NBUF = 3
ZPT = 640
ZLAST = N - 15 * ZPT
ZROWS = 80

EBLK = 16000
EGRID2 = EH // EBLK


def _wfilt_body(a1_ref, a2_ref, w1_ref, w2_ref, w1aug_ref, we2a_ref,
                x_ref, wl1_ref, out_ref, h_ref):
    step = CUTOFF / (NG - 1)
    coeff = -0.5 / step ** 2
    offs = lax.broadcasted_iota(jnp.int32, (NGA, 1), 0).astype(jnp.float32) * step

    def half(attr_row, w_row):
        attr_row = attr_row.reshape(1, EBLK)
        w_row = w_row.reshape(1, EBLK)
        cc = 0.5 * (jnp.cos(w_row * (math.pi / CUTOFF)) + 1.0)
        dist = attr_row - offs
        smeared = jnp.exp(coeff * dist * dist)
        ridx = lax.broadcasted_iota(jnp.int32, (NGA, EBLK), 0)
        smeared = jnp.where(ridx == NG, 1.0, smeared)
        smeared = jnp.where(ridx > NG, 0.0, smeared)
        smc = smeared * cc
        hgt = lax.dot_general(w1aug_ref[...], smc,
                              (((1,), (0,)), ((), ())),
                              preferred_element_type=jnp.float32)
        hgt = jnp.maximum(hgt, 0.0)
        ccpad = jnp.where(
            lax.broadcasted_iota(jnp.int32, (8, EBLK), 0) == 0, cc, 0.0)
        aug = jnp.concatenate([hgt, ccpad], axis=0)
        return lax.dot_general(aug, we2a_ref[...],
                               (((0,), (0,)), ((), ())),
                               preferred_element_type=jnp.float32)

    wfa = half(a1_ref[...], w1_ref[...])
    wfb = half(a2_ref[...], w2_ref[...])
    out_ref[...] = jnp.concatenate([wfa, wfb], axis=1)

    @pl.when(pl.program_id(0) == 0)
    def _h():
        h_ref[...] = jnp.dot(x_ref[...], wl1_ref[...],
                             preferred_element_type=jnp.float32)


_wfilt_call = pl.pallas_call(
    _wfilt_body,
    grid=(EGRID2,),
    in_specs=[
        pl.BlockSpec((1, 1, EBLK), lambda i: (i, 0, 0)),
        pl.BlockSpec((1, 1, EBLK), lambda i: (i + EGRID2, 0, 0)),
        pl.BlockSpec((1, 1, EBLK), lambda i: (i, 0, 0)),
        pl.BlockSpec((1, 1, EBLK), lambda i: (i + EGRID2, 0, 0)),
        pl.BlockSpec((NF, NGA), lambda i: (0, 0)),
        pl.BlockSpec((NF + 8, NF), lambda i: (0, 0)),
        pl.BlockSpec((N, D), lambda i: (0, 0)),
        pl.BlockSpec((D, NFP), lambda i: (0, 0)),
    ],
    out_specs=[
        pl.BlockSpec((EBLK, NFP), lambda i: (i, 0)),
        pl.BlockSpec((N, NFP), lambda i: (0, 0)),
    ],
    out_shape=[
        jax.ShapeDtypeStruct((EH, NFP), jnp.float32),
        jax.ShapeDtypeStruct((N, NFP), jnp.float32),
    ],
    compiler_params=pltpu.CompilerParams(
        fuse_transposed_lhs_in_matmul=True),
)


def _sc_body(h_hbm, wf_hbm, src_hbm, dst_hbm, out_hbm,
             agg_sh, src_v, dst_v, wf_v, rows_v, sem_in, sem_g,
             sem_s):
    cid = lax.axis_index("c")
    sid = lax.axis_index("s")
    wid = sid * NC + cid
    rb0 = wid * RPW

    def _zrow(i, carry):
        for j in range(NFP // L):
            rows_v[0, i, pl.ds(j * L, L)] = jnp.zeros((L,), jnp.float32)
        return carry
    lax.fori_loop(0, ZROWS, _zrow, 0)
    nrep = jnp.where(sid < NS - 1, ZPT // ZROWS, ZLAST // ZROWS)

    def _zcopy(k, carry):
        zbase = pl.multiple_of(sid * ZPT + k * ZROWS, 8)
        pltpu.async_copy(rows_v.at[0], agg_sh.at[pl.ds(zbase, ZROWS)],
                         sem_in)
        return carry
    lax.fori_loop(0, nrep, _zcopy, 0)

    def _zwait(k, carry):
        zbase = pl.multiple_of(sid * ZPT + k * ZROWS, 8)
        pltpu.make_async_copy(rows_v.at[0],
                              agg_sh.at[pl.ds(zbase, ZROWS)],
                              sem_in).wait()
        return carry
    lax.fori_loop(0, nrep, _zwait, 0)
    plsc.subcore_barrier()

    def _front_copies(c):
        b = c % NBUF
        rbase = pl.multiple_of(rb0 + c * CR, 8)
        return (
            (src_hbm.at[pl.ds(rbase, CR)], src_v.at[b, pl.ds(0, CR)]),
            (src_hbm.at[pl.ds(EH + rbase, CR)], src_v.at[b, pl.ds(CR, CR)]),
            (dst_hbm.at[pl.ds(rbase, CR)], dst_v.at[b, pl.ds(0, CR)]),
            (dst_hbm.at[pl.ds(EH + rbase, CR)], dst_v.at[b, pl.ds(CR, CR)]),
            (wf_hbm.at[pl.ds(rbase, CR)], wf_v.at[b]),
        )

    def _front(c):
        for s, d in _front_copies(c):
            pltpu.async_copy(s, d, sem_in)

    def _front_wait(c):
        for s, d in _front_copies(c):
            pltpu.make_async_copy(s, d, sem_in).wait()

    def _gather(c):
        b = c % NBUF
        pltpu.async_copy(h_hbm.at[src_v.at[b]], rows_v.at[b], sem_g)

    def _gather_wait(c):
        b = c % NBUF
        pltpu.make_async_copy(h_hbm.at[src_v.at[b]], rows_v.at[b],
                              sem_g).wait()

    def _scatter(c):
        b = c % NBUF
        pltpu.async_copy(rows_v.at[b], agg_sh.at[dst_v.at[b]], sem_s,
                         add=True)

    def _scatter_wait(c):
        b = c % NBUF
        pltpu.make_async_copy(rows_v.at[b], agg_sh.at[dst_v.at[b]],
                              sem_s).wait()

    _front(0)
    _front(1)
    _front_wait(0)
    _gather(0)

    def _step(c, carry):
        b = c % NBUF
        _gather_wait(c)

        @pl.when(c + 1 < NCHUNK)
        def _():
            _front_wait(c + 1)

            @pl.when(c >= 2)
            def _():
                _scatter_wait(c - 2)
            _gather(c + 1)

        @pl.when(c + 2 < NCHUNK)
        def _():
            _front(c + 2)

        def _mul(i, icarry):
            for j in range(NF // L):
                s = pl.ds(j * L, L)
                s2 = pl.ds(NF + j * L, L)
                rows_v[b, i, s] = rows_v[b, i, s] * wf_v[b, i, s]
                rows_v[b, CR + i, s] = rows_v[b, CR + i, s] * wf_v[b, i, s2]
            return icarry
        lax.fori_loop(0, CR, _mul, 0, unroll=4)

        _scatter(c)
        return carry
    lax.fori_loop(0, NCHUNK, _step, 0)
    _scatter_wait(NCHUNK - 2)
    _scatter_wait(NCHUNK - 1)

    plsc.subcore_barrier()

    zbase = pl.multiple_of(sid * ZPT, 8)
    obase = pl.multiple_of(cid * N + sid * ZPT, 8)

    @pl.when(sid < NS - 1)
    def _copy_out():
        pltpu.sync_copy(agg_sh.at[pl.ds(zbase, ZPT)],
                        out_hbm.at[pl.ds(obase, ZPT)])

    @pl.when(sid == NS - 1)
    def _copy_out_last():
        pltpu.sync_copy(agg_sh.at[pl.ds(zbase, ZLAST)],
                        out_hbm.at[pl.ds(obase, ZLAST)])


_sc_call = functools.partial(
    pl.kernel,
    mesh=plsc.VectorSubcoreMesh(core_axis_name="c", subcore_axis_name="s"),
    out_type=jax.ShapeDtypeStruct((NC * N, NFP), jnp.float32),
    scratch_types=[
        pltpu.VMEM_SHARED((N, NFP), jnp.float32),
        pltpu.VMEM((NBUF, CHUNK), jnp.int32),
        pltpu.VMEM((NBUF, CHUNK), jnp.int32),
        pltpu.VMEM((NBUF, CR, NFP), jnp.float32),
        pltpu.VMEM((NBUF, CHUNK, NFP), jnp.float32),
        pltpu.SemaphoreType.DMA,
        pltpu.SemaphoreType.DMA,
        pltpu.SemaphoreType.DMA,
    ],
)(_sc_body)


def _final_body(agg_ref, x_ref, w_ref, b_ref, out_ref):
    aggsum = agg_ref[:N, :NF] + agg_ref[N:, :NF]
    y = jnp.dot(aggsum, w_ref[...], preferred_element_type=jnp.float32)
    y = jnp.maximum(y + b_ref[...], 0.0)
    out_ref[...] = x_ref[...] + y


_final_call = pl.pallas_call(
    _final_body,
    out_shape=jax.ShapeDtypeStruct((N, D), jnp.float32),
)


def kernel(x, edge_index, edge_attr, edge_weight, W_lin1, W_e1, b_e1,
           W_e2, b_e2, W_lin2, b_lin2):
    src = edge_index[0].astype(jnp.int32)
    dst = edge_index[1].astype(jnp.int32)
    attr2 = edge_attr.reshape(2 * EGRID2, 1, EBLK)
    wgt2 = edge_weight.reshape(2 * EGRID2, 1, EBLK)
    w1p = jnp.concatenate([W_lin1, jnp.zeros((D, NFP - NF), jnp.float32)],
                          axis=1)
    w1aug = jnp.concatenate(
        [W_e1.T, b_e1.reshape(NF, 1),
         jnp.zeros((NF, NGA - NG - 1), jnp.float32)], axis=1)
    we2a = jnp.concatenate(
        [W_e2, b_e2.reshape(1, NF), jnp.zeros((7, NF), jnp.float32)],
        axis=0)
    wfilt, h = _wfilt_call(attr2, attr2, wgt2, wgt2, w1aug, we2a, x, w1p)
    agg2 = _sc_call(h, wfilt, src, dst)
    return _final_call(agg2, x, W_lin2, b_lin2.reshape(1, D))
